# Initial kernel scaffold; baseline (speedup 1.0000x reference)
import jax
import jax.numpy as jnp
from jax.experimental import pallas as pl


def kernel(x, dist_feat, atom_idx, ele_idx, W1, b1, gamma, beta):
    raise NotImplementedError("write your pallas kernel here")


# speedup vs baseline: 2.7558x; 2.7558x over previous
import functools

import jax
import jax.numpy as jnp
from jax import lax
from jax.experimental import pallas as pl
from jax.experimental.pallas import tpu as pltpu

N_ROWS = 800000
N_AE = 32
N_DE = 16
N_SEG_ATOM = 50000
N_SEG_ELE = 100

B = 1280
NB = N_ROWS // B
W = 128
TR = 50256
TE = 128
FS = 40


def _phase1_body(i, lo_ref, hi_ref, x_ref, aidx_ref, eidx_ref, aacc_ref, eacc_ref):
    b = i
    aidx = aidx_ref[0]
    eidx = eidx_ref[0]
    xa = x_ref[:, :N_AE]
    xe = x_ref[:, N_AE:]
    ones8 = jnp.ones((B, 8), jnp.float32)
    xa40 = jnp.concatenate([xa, ones8], axis=1)
    xe40 = jnp.concatenate([xe, ones8], axis=1)

    iota_e = lax.broadcasted_iota(jnp.int32, (TE, B), 0)
    ohe = (iota_e == eidx).astype(jnp.float32)
    eacc_ref[...] += jnp.dot(ohe, xe40, preferred_element_type=jnp.float32)

    lo = lo_ref[b]
    hi = hi_ref[b]
    base = (lo // 8) * 8
    nwin = (hi - base) // W + 1
    iota_a = lax.broadcasted_iota(jnp.int32, (W, B), 0)

    def wloop(k, _):
        ws = base + k * W
        oh = ((iota_a + ws) == aidx).astype(jnp.float32)
        contrib = jnp.dot(oh, xa40, preferred_element_type=jnp.float32)
        aacc_ref[pl.ds(ws, W), :] += contrib
        return 0

    lax.fori_loop(0, nwin, wloop, 0)


def _phase2_body(i, lo_ref, hi_ref, dist_ref, aidx_ref, eidx_ref, w1_ref, b1_ref,
                 h_ref, stats_ref, aacc_ref, eacc_ref):
    b = i - NB
    aidx = aidx_ref[0]
    eidx = eidx_ref[0]

    ecnt = jnp.maximum(eacc_ref[:, N_AE:N_AE + 1], 1.0)
    pe_tab = jnp.maximum(eacc_ref[:, :N_AE] / ecnt, 0.0)
    iota_eg = lax.broadcasted_iota(jnp.int32, (B, TE), 1)
    ohe = (iota_eg == eidx.reshape(B, 1)).astype(jnp.float32)
    pe = jnp.dot(ohe, pe_tab, preferred_element_type=jnp.float32)

    lo = lo_ref[b]
    hi = hi_ref[b]
    base = (lo // 8) * 8
    nwin = (hi - base) // W + 1
    iota_ag = lax.broadcasted_iota(jnp.int32, (B, W), 1)
    aidx_col = aidx.reshape(B, 1)

    def wloop(k, pa):
        ws = base + k * W
        win = aacc_ref[pl.ds(ws, W), :]
        cnt = jnp.maximum(win[:, N_AE:N_AE + 1], 1.0)
        ptab = jnp.maximum(win[:, :N_AE] / cnt, 0.0)
        oh = ((iota_ag + ws) == aidx_col).astype(jnp.float32)
        return pa + jnp.dot(oh, ptab, preferred_element_type=jnp.float32)

    pa = lax.fori_loop(0, nwin, wloop, jnp.zeros((B, N_AE), jnp.float32))

    c = jnp.concatenate([dist_ref[...], pa, pe], axis=1)
    hb = jnp.dot(c, w1_ref[...], preferred_element_type=jnp.float32) + b1_ref[...]
    h_ref[...] = hb

    @pl.when(i == NB)
    def _():
        stats_ref[...] = jnp.zeros((8, 64), jnp.float32)

    s1 = jnp.sum(hb, axis=0)
    s2 = jnp.sum(hb * hb, axis=0)
    stats_ref[0:1, :] += s1[None, :]
    stats_ref[1:2, :] += s2[None, :]


def _k1_body(lo_ref, hi_ref, x_ref, dist_ref, aidx_ref, eidx_ref, w1_ref, b1_ref,
             h_ref, stats_ref, aacc_ref, eacc_ref):
    i = pl.program_id(0)

    @pl.when(i == 0)
    def _():
        aacc_ref[...] = jnp.zeros((TR, FS), jnp.float32)
        eacc_ref[...] = jnp.zeros((TE, FS), jnp.float32)

    @pl.when(i < NB)
    def _():
        _phase1_body(i, lo_ref, hi_ref, x_ref, aidx_ref, eidx_ref, aacc_ref, eacc_ref)

    @pl.when(i >= NB)
    def _():
        _phase2_body(i, lo_ref, hi_ref, dist_ref, aidx_ref, eidx_ref, w1_ref, b1_ref,
                     h_ref, stats_ref, aacc_ref, eacc_ref)


def _k2_body(h_ref, x_ref, stats_ref, gamma_ref, beta_ref, out_ref):
    inv_n = 1.0 / N_ROWS
    mu = stats_ref[0:1, :] * inv_n
    ex2 = stats_ref[1:2, :] * inv_n
    var = ex2 - mu * mu
    inv = lax.rsqrt(var + 1e-5)
    scale = gamma_ref[...] * inv
    shift = beta_ref[...] - mu * scale
    out_ref[...] = jnp.maximum(h_ref[...] * scale + shift + x_ref[...], 0.0)


@jax.jit
def kernel(x, dist_feat, atom_idx, ele_idx, W1, b1, gamma, beta):
    aidx = atom_idx.astype(jnp.int32)
    eidx = ele_idx.astype(jnp.int32)
    lo = aidx[::B]
    hi = aidx[B - 1::B]
    aidx3 = aidx.reshape(NB, 1, B)
    eidx3 = eidx.reshape(NB, 1, B)
    b1r = b1.reshape(1, 64)

    grid1 = pltpu.PrefetchScalarGridSpec(
        num_scalar_prefetch=2,
        grid=(2 * NB,),
        in_specs=[
            pl.BlockSpec((B, 64), lambda i, lo, hi: (jnp.where(i < NB, i, NB - 1), 0)),
            pl.BlockSpec((B, N_DE), lambda i, lo, hi: (jnp.where(i < NB, 0, i - NB), 0)),
            pl.BlockSpec((1, 1, B), lambda i, lo, hi: (jnp.where(i < NB, i, i - NB), 0, 0)),
            pl.BlockSpec((1, 1, B), lambda i, lo, hi: (jnp.where(i < NB, i, i - NB), 0, 0)),
            pl.BlockSpec((80, 64), lambda i, lo, hi: (0, 0)),
            pl.BlockSpec((1, 64), lambda i, lo, hi: (0, 0)),
        ],
        out_specs=[
            pl.BlockSpec((B, 64), lambda i, lo, hi: (jnp.where(i < NB, 0, i - NB), 0)),
            pl.BlockSpec((8, 64), lambda i, lo, hi: (0, 0)),
        ],
        scratch_shapes=[
            pltpu.VMEM((TR, FS), jnp.float32),
            pltpu.VMEM((TE, FS), jnp.float32),
        ],
    )
    h, stats = pl.pallas_call(
        _k1_body,
        grid_spec=grid1,
        out_shape=[
            jax.ShapeDtypeStruct((N_ROWS, 64), jnp.float32),
            jax.ShapeDtypeStruct((8, 64), jnp.float32),
        ],
        compiler_params=pltpu.CompilerParams(
            dimension_semantics=("arbitrary",),
        ),
    )(lo, hi, x, dist_feat, aidx3, eidx3, W1, b1r)

    out = pl.pallas_call(
        _k2_body,
        grid=(NB,),
        in_specs=[
            pl.BlockSpec((B, 64), lambda i: (i, 0)),
            pl.BlockSpec((B, 64), lambda i: (i, 0)),
            pl.BlockSpec((8, 64), lambda i: (0, 0)),
            pl.BlockSpec((1, 64), lambda i: (0, 0)),
            pl.BlockSpec((1, 64), lambda i: (0, 0)),
        ],
        out_specs=pl.BlockSpec((B, 64), lambda i: (i, 0)),
        out_shape=jax.ShapeDtypeStruct((N_ROWS, 64), jnp.float32),
        compiler_params=pltpu.CompilerParams(
            dimension_semantics=("arbitrary",),
        ),
    )(h, x, stats, gamma.reshape(1, 64), beta.reshape(1, 64))
    return out
